# baseline (speedup 1.0000x reference)
import jax
import jax.numpy as jnp
import numpy as np
from jax.experimental import pallas as pl
from jax.experimental.pallas import tpu as pltpu

_B = 8192
_D = 128
_T = 8
_K = 2
_L = 3
_NF = 8
_VR = 16
_NP = _VR * _VR
_BB = 2048

_O_WI = 0
_O_BI = 32
_O_BR = 35
_O_LNG = 38
_O_LNB = 41
_O_BS1 = 44
_O_BS2 = 45
_O_BD1 = 46
_O_BD2 = 47
_O_WR = 48
_O_W1 = _O_WR + _L * _D
_O_B1 = _O_W1 + _L * _T * _D
_O_W2 = _O_B1 + _L * _T
_O_B2 = _O_W2 + _L * _T * _D
_O_WS1 = _O_B2 + _L * _T
_O_WS2 = _O_WS1 + _D
_O_WD1 = _O_WS2 + _D
_O_WD2 = _O_WD1 + _D
_ROWS = _O_WD2 + _D


def _gelu(x):
    return x * 0.5 * (1.0 + jax.lax.erf(x * np.float32(1.0 / np.sqrt(2.0))))


def _ln(x, g, b):
    m = jnp.mean(x, axis=-1, keepdims=True)
    xc = x - m
    v = jnp.mean(xc * xc, axis=-1, keepdims=True)
    return xc * jax.lax.rsqrt(v + 1e-5) * g + b


def _net_kernel(a_ref, b_ref, w_ref,
                sum_ref, diff_ref, rout_ref, aux_ref,
                tab_ref, ps_ref, oh_ref, cnt_ref):
    i = pl.program_id(0)
    nsteps = pl.num_programs(0)

    @pl.when(i == 0)
    def _build_tables():
        cnt_ref[...] = jnp.zeros_like(cnt_ref)
        pair = jax.lax.broadcasted_iota(jnp.int32, (_NP, 1), 0)
        pa = (pair // _VR).astype(jnp.float32)
        pb = (pair % _VR).astype(jnp.float32)
        fexp = jax.lax.broadcasted_iota(jnp.int32, (1, _NF), 1)
        freqs = jnp.exp2(fexp.astype(jnp.float32)) * np.float32(
            2.0 * np.pi / _VR)
        av = pa * freqs
        bv = pb * freqs
        x0 = jnp.concatenate(
            [jnp.sin(av), jnp.cos(av), jnp.sin(bv), jnp.cos(bv)], axis=1)

        x = jnp.dot(x0, w_ref[_O_WI:_O_WI + 32],
                    preferred_element_type=jnp.float32)
        x = _gelu(_ln(x + w_ref[_O_BI], w_ref[_O_BI + 1], w_ref[_O_BI + 2]))

        ti = jax.lax.broadcasted_iota(jnp.int32, (_NP, _T), 1)
        i1 = i2 = None
        for l in range(_L):
            wr = w_ref[_O_WR + l * _D:_O_WR + (l + 1) * _D, 0:_T]
            logits = jnp.dot(x, wr, preferred_element_type=jnp.float32)
            logits = logits + w_ref[_O_BR + l, 0:_T]
            p = logits - jnp.max(logits, axis=1, keepdims=True)
            p = jnp.exp(p)
            p = p / jnp.sum(p, axis=1, keepdims=True)

            m1 = jnp.max(p, axis=1, keepdims=True)
            i1 = jnp.min(jnp.where(p == m1, ti, _T), axis=1, keepdims=True)
            p_rest = jnp.where(ti == i1, -jnp.inf, p)
            m2 = jnp.max(p_rest, axis=1, keepdims=True)
            i2 = jnp.min(jnp.where(p_rest == m2, ti, _T), axis=1,
                         keepdims=True)
            denom = 1.0 / (m1 + m2 + 1e-9)
            oh = ((ti == i1) | (ti == i2)).astype(jnp.float32)
            gates_full = jnp.where(ti == i1, m1 * denom, 0.0) + jnp.where(
                ti == i2, m2 * denom, 0.0)

            ps_ref[:, l * _T:(l + 1) * _T] = p
            oh_ref[:, l * _T:(l + 1) * _T] = oh

            out = jnp.zeros((_NP, _D), jnp.float32)
            for t in range(_T):
                r1 = _O_W1 + (l * _T + t) * _D
                r2 = _O_W2 + (l * _T + t) * _D
                h = jnp.dot(x, w_ref[r1:r1 + _D],
                            preferred_element_type=jnp.float32)
                h = _gelu(h + w_ref[_O_B1 + l * _T + t])
                eo = jnp.dot(h, w_ref[r2:r2 + _D],
                             preferred_element_type=jnp.float32)
                eo = eo + w_ref[_O_B2 + l * _T + t]
                out = out + gates_full[:, t][:, None] * eo
            x = _ln(x + out, w_ref[_O_LNG + l], w_ref[_O_LNB + l])

        sl = jnp.dot(_gelu(jnp.dot(x, w_ref[_O_WS1:_O_WS1 + _D],
                                   preferred_element_type=jnp.float32)
                           + w_ref[_O_BS1]),
                     w_ref[_O_WS2:_O_WS2 + _D],
                     preferred_element_type=jnp.float32)
        dl = jnp.dot(_gelu(jnp.dot(x, w_ref[_O_WD1:_O_WD1 + _D],
                                   preferred_element_type=jnp.float32)
                           + w_ref[_O_BD1]),
                     w_ref[_O_WD2:_O_WD2 + _D],
                     preferred_element_type=jnp.float32)
        tab = jnp.concatenate(
            [(sl + w_ref[_O_BS2])[:, 0:5],
             (dl + w_ref[_O_BD2])[:, 0:6],
             i1.astype(jnp.float32), i2.astype(jnp.float32),
             jnp.zeros((_NP, _D - 13), jnp.float32)], axis=1)
        tab_ref[...] = tab

    pair_t = (a_ref[...] * _VR + b_ref[...])[:, None]
    onehot = (pair_t == jax.lax.broadcasted_iota(
        jnp.int32, (_BB, _NP), 1)).astype(jnp.float32)
    g = jnp.dot(onehot, tab_ref[...], preferred_element_type=jnp.float32)
    sum_ref[...] = g[:, 0:5]
    diff_ref[...] = g[:, 5:11]
    rout_ref[...] = g[:, 11:13].astype(jnp.int32)
    cnt_ref[...] += jnp.sum(onehot, axis=0, keepdims=True)

    @pl.when(i == nsteps - 1)
    def _fin():
        ps_sum = jnp.dot(cnt_ref[...], ps_ref[...],
                         preferred_element_type=jnp.float32)
        ls_sum = jnp.dot(cnt_ref[...], oh_ref[...],
                         preferred_element_type=jnp.float32)
        scale = np.float32(_T) / np.float32(_B * _B)
        aux_ref[...] = jnp.sum(ps_sum * ls_sum, keepdims=True).reshape(
            1, 1) * scale


def _pack(p):
    def padw(x):
        return jnp.pad(x, ((0, 0), (0, _D - x.shape[1])))

    return jnp.concatenate([
        p["Wi"],
        p["bi"][None, :], p["ln_in_g"][None, :], p["ln_in_b"][None, :],
        padw(p["br"]),
        p["ln_g"], p["ln_b"],
        jnp.pad(p["bs1"], (0, _D - 64))[None, :],
        jnp.pad(p["bs2"], (0, _D - 5))[None, :],
        jnp.pad(p["bd1"], (0, _D - 64))[None, :],
        jnp.pad(p["bd2"], (0, _D - 6))[None, :],
        padw(p["Wr"].reshape(_L * _D, _T)),
        p["W1"].reshape(_L * _T * _D, _D),
        p["b1"].reshape(_L * _T, _D),
        p["W2"].reshape(_L * _T * _D, _D),
        p["b2"].reshape(_L * _T, _D),
        padw(p["Ws1"]),
        jnp.pad(p["Ws2"], ((0, _D - 64), (0, _D - 5))),
        padw(p["Wd1"]),
        jnp.pad(p["Wd2"], ((0, _D - 64), (0, _D - 6))),
    ], axis=0)


@jax.jit
def _run(a, b, params):
    nb = _B // _BB
    wpack = _pack(params)

    in_specs = [
        pl.BlockSpec((_BB,), lambda i: (i,)),
        pl.BlockSpec((_BB,), lambda i: (i,)),
        pl.BlockSpec((_ROWS, _D), lambda i: (0, 0)),
    ]
    out_specs = [
        pl.BlockSpec((_BB, 5), lambda i: (i, 0)),
        pl.BlockSpec((_BB, 6), lambda i: (i, 0)),
        pl.BlockSpec((_BB, _K), lambda i: (i, 0)),
        pl.BlockSpec((1, 1), lambda i: (0, 0)),
    ]
    out_shape = [
        jax.ShapeDtypeStruct((_B, 5), jnp.float32),
        jax.ShapeDtypeStruct((_B, 6), jnp.float32),
        jax.ShapeDtypeStruct((_B, _K), jnp.int32),
        jax.ShapeDtypeStruct((1, 1), jnp.float32),
    ]
    sl, dl, rout, aux = pl.pallas_call(
        _net_kernel,
        grid=(nb,),
        in_specs=in_specs,
        out_specs=out_specs,
        out_shape=out_shape,
        scratch_shapes=[
            pltpu.VMEM((_NP, _D), jnp.float32),
            pltpu.VMEM((_NP, _L * _T), jnp.float32),
            pltpu.VMEM((_NP, _L * _T), jnp.float32),
            pltpu.VMEM((1, _NP), jnp.float32),
        ],
    )(a.astype(jnp.int32), b.astype(jnp.int32), wpack)
    return sl, dl, rout, aux[0, 0]


def kernel(a, b, params):
    return _run(a, b, params)


# speedup vs baseline: 4.1639x; 4.1639x over previous
import jax
import jax.numpy as jnp
import numpy as np
from jax.experimental import pallas as pl
from jax.experimental.pallas import tpu as pltpu

_B = 8192
_D = 128
_T = 8
_K = 2
_L = 3
_NF = 8
_VR = 16
_NP = _VR * _VR
_BB = 2048

_DNT = (((1,), (1,)), ((), ()))


def _gelu(x):
    return x * 0.5 * (1.0 + jax.lax.erf(x * np.float32(1.0 / np.sqrt(2.0))))


def _ln(x, g, b):
    m = jnp.mean(x, axis=-1, keepdims=True)
    xc = x - m
    v = jnp.mean(xc * xc, axis=-1, keepdims=True)
    return xc * jax.lax.rsqrt(v + 1e-5) * g + b


def _net_kernel(a_ref, b_ref, Wi_ref, bi_ref, lig_ref, lib_ref,
                WrT_ref, br_ref, W1_ref, b1_ref, W2_ref, b2_ref,
                lng_ref, lnb_ref, Ws1T_ref, bs1_ref, Ws2T_ref, bs2_ref,
                Wd1T_ref, bd1_ref, Wd2T_ref, bd2_ref,
                sumT_ref, diffT_ref, routT_ref, aux_ref,
                tabT_ref, ps_ref, oh_ref, cnt_ref):
    i = pl.program_id(0)
    nsteps = pl.num_programs(0)

    @pl.when(i == 0)
    def _build_tables():
        cnt_ref[...] = jnp.zeros_like(cnt_ref)
        pair = jax.lax.broadcasted_iota(jnp.int32, (_NP, 1), 0)
        pa = (pair // _VR).astype(jnp.float32)
        pb = (pair % _VR).astype(jnp.float32)
        fexp = jax.lax.broadcasted_iota(jnp.int32, (1, _NF), 1)
        freqs = jnp.exp2(fexp.astype(jnp.float32)) * np.float32(
            2.0 * np.pi / _VR)
        av = pa * freqs
        bv = pb * freqs
        x0 = jnp.concatenate(
            [jnp.sin(av), jnp.cos(av), jnp.sin(bv), jnp.cos(bv)], axis=1)

        x = jnp.dot(x0, Wi_ref[...], preferred_element_type=jnp.float32)
        x = _gelu(_ln(x + bi_ref[...], lig_ref[...], lib_ref[...]))

        ti = jax.lax.broadcasted_iota(jnp.int32, (_NP, _T), 1)
        i1 = i2 = None
        for l in range(_L):
            logits = jax.lax.dot_general(
                x, WrT_ref[l], _DNT, preferred_element_type=jnp.float32)
            logits = logits + br_ref[l]
            p = logits - jnp.max(logits, axis=1, keepdims=True)
            p = jnp.exp(p)
            p = p / jnp.sum(p, axis=1, keepdims=True)

            m1 = jnp.max(p, axis=1, keepdims=True)
            i1 = jnp.min(jnp.where(p == m1, ti, _T), axis=1, keepdims=True)
            p_rest = jnp.where(ti == i1, -jnp.inf, p)
            m2 = jnp.max(p_rest, axis=1, keepdims=True)
            i2 = jnp.min(jnp.where(p_rest == m2, ti, _T), axis=1,
                         keepdims=True)
            denom = 1.0 / (m1 + m2 + 1e-9)
            oh = ((ti == i1) | (ti == i2)).astype(jnp.float32)
            gates_full = jnp.where(ti == i1, m1 * denom, 0.0) + jnp.where(
                ti == i2, m2 * denom, 0.0)

            ps_ref[:, l * _T:(l + 1) * _T] = p
            oh_ref[:, l * _T:(l + 1) * _T] = oh

            out = jnp.zeros((_NP, _D), jnp.float32)
            for t in range(_T):
                h = jnp.dot(x, W1_ref[l, t],
                            preferred_element_type=jnp.float32)
                h = _gelu(h + b1_ref[l, t])
                eo = jnp.dot(h, W2_ref[l, t],
                             preferred_element_type=jnp.float32)
                eo = eo + b2_ref[l, t]
                out = out + gates_full[:, t][:, None] * eo
            x = _ln(x + out, lng_ref[l], lnb_ref[l])

        hs = _gelu(jax.lax.dot_general(
            x, Ws1T_ref[...], _DNT, preferred_element_type=jnp.float32)
            + bs1_ref[...])
        sl = jax.lax.dot_general(
            hs, Ws2T_ref[...], _DNT, preferred_element_type=jnp.float32)
        hd = _gelu(jax.lax.dot_general(
            x, Wd1T_ref[...], _DNT, preferred_element_type=jnp.float32)
            + bd1_ref[...])
        dl = jax.lax.dot_general(
            hd, Wd2T_ref[...], _DNT, preferred_element_type=jnp.float32)
        tab = jnp.concatenate(
            [sl + bs2_ref[...], dl + bd2_ref[...],
             i1.astype(jnp.float32), i2.astype(jnp.float32),
             jnp.zeros((_NP, _D - 13), jnp.float32)], axis=1)
        tabT_ref[...] = tab.T

    pair_row = (a_ref[...] * _VR + b_ref[...])[None, :]
    onehotT = (pair_row == jax.lax.broadcasted_iota(
        jnp.int32, (_NP, _BB), 0)).astype(jnp.float32)
    gT = jnp.dot(tabT_ref[0:16], onehotT, preferred_element_type=jnp.float32)
    sumT_ref[...] = gT[0:5, :]
    diffT_ref[...] = gT[5:11, :]
    routT_ref[...] = gT[11:13, :].astype(jnp.int32)
    cnt_ref[...] += jnp.sum(onehotT, axis=1, keepdims=True)

    @pl.when(i == nsteps - 1)
    def _fin():
        dnc = (((0,), (0,)), ((), ()))
        ps_sum = jax.lax.dot_general(
            cnt_ref[...], ps_ref[...], dnc,
            preferred_element_type=jnp.float32)
        ls_sum = jax.lax.dot_general(
            cnt_ref[...], oh_ref[...], dnc,
            preferred_element_type=jnp.float32)
        scale = np.float32(_T) / np.float32(_B * _B)
        aux_ref[...] = jnp.sum(ps_sum * ls_sum, keepdims=True).reshape(
            1, 1) * scale


@jax.jit
def _run(a, b, params):
    nb = _B // _BB
    p = params

    full = lambda s: pl.BlockSpec(s, lambda i: (0,) * len(s))
    in_specs = [
        pl.BlockSpec((_BB,), lambda i: (i,)),
        pl.BlockSpec((_BB,), lambda i: (i,)),
        full((4 * _NF, _D)), full((_D,)), full((_D,)), full((_D,)),
        full((_L, _T, _D)), full((_L, _T)),
        full((_L, _T, _D, _D)), full((_L, _T, _D)),
        full((_L, _T, _D, _D)), full((_L, _T, _D)),
        full((_L, _D)), full((_L, _D)),
        full((_D // 2, _D)), full((_D // 2,)),
        full((5, _D // 2)), full((5,)),
        full((_D // 2, _D)), full((_D // 2,)),
        full((6, _D // 2)), full((6,)),
    ]
    out_specs = [
        pl.BlockSpec((5, _BB), lambda i: (0, i)),
        pl.BlockSpec((6, _BB), lambda i: (0, i)),
        pl.BlockSpec((_K, _BB), lambda i: (0, i)),
        pl.BlockSpec((1, 1), lambda i: (0, 0)),
    ]
    out_shape = [
        jax.ShapeDtypeStruct((5, _B), jnp.float32),
        jax.ShapeDtypeStruct((6, _B), jnp.float32),
        jax.ShapeDtypeStruct((_K, _B), jnp.int32),
        jax.ShapeDtypeStruct((1, 1), jnp.float32),
    ]
    slT, dlT, routT, aux = pl.pallas_call(
        _net_kernel,
        grid=(nb,),
        in_specs=in_specs,
        out_specs=out_specs,
        out_shape=out_shape,
        scratch_shapes=[
            pltpu.VMEM((_D, _NP), jnp.float32),
            pltpu.VMEM((_NP, _L * _T), jnp.float32),
            pltpu.VMEM((_NP, _L * _T), jnp.float32),
            pltpu.VMEM((_NP, 1), jnp.float32),
        ],
    )(a.astype(jnp.int32), b.astype(jnp.int32),
      p["Wi"], p["bi"], p["ln_in_g"], p["ln_in_b"],
      jnp.swapaxes(p["Wr"], 1, 2), p["br"],
      p["W1"], p["b1"], p["W2"], p["b2"],
      p["ln_g"], p["ln_b"],
      p["Ws1"].T, p["bs1"], p["Ws2"].T, p["bs2"],
      p["Wd1"].T, p["bd1"], p["Wd2"].T, p["bd2"])
    return slT.T, dlT.T, routT.T, aux[0, 0]


def kernel(a, b, params):
    return _run(a, b, params)


# speedup vs baseline: 4.6037x; 1.1056x over previous
import jax
import jax.numpy as jnp
import numpy as np
from jax.experimental import pallas as pl
from jax.experimental.pallas import tpu as pltpu

_B = 8192
_D = 128
_T = 8
_K = 2
_L = 3
_NF = 8
_VR = 16
_NP = _VR * _VR
_BB = 8192

_DNT = (((1,), (1,)), ((), ()))


def _gelu(x):
    return x * 0.5 * (1.0 + jax.lax.erf(x * np.float32(1.0 / np.sqrt(2.0))))


def _ln(x, g, b):
    m = jnp.mean(x, axis=-1, keepdims=True)
    xc = x - m
    v = jnp.mean(xc * xc, axis=-1, keepdims=True)
    return xc * jax.lax.rsqrt(v + 1e-5) * g + b


def _net_kernel(a_ref, b_ref, Wi_ref, bi_ref, lig_ref, lib_ref,
                WrT_ref, br_ref, W1_ref, b1_ref, W2_ref, b2_ref,
                lng_ref, lnb_ref, Ws1T_ref, bs1_ref, Ws2T_ref, bs2_ref,
                Wd1T_ref, bd1_ref, Wd2T_ref, bd2_ref,
                sumT_ref, diffT_ref, routT_ref, aux_ref,
                tabT_ref, ps_ref, oh_ref, cnt_ref):
    i = pl.program_id(0)
    nsteps = pl.num_programs(0)

    @pl.when(i == 0)
    def _build_tables():
        cnt_ref[...] = jnp.zeros_like(cnt_ref)
        pair = jax.lax.broadcasted_iota(jnp.int32, (_NP, 1), 0)
        pa = (pair // _VR).astype(jnp.float32)
        pb = (pair % _VR).astype(jnp.float32)
        fexp = jax.lax.broadcasted_iota(jnp.int32, (1, _NF), 1)
        freqs = jnp.exp2(fexp.astype(jnp.float32)) * np.float32(
            2.0 * np.pi / _VR)
        av = pa * freqs
        bv = pb * freqs
        x0 = jnp.concatenate(
            [jnp.sin(av), jnp.cos(av), jnp.sin(bv), jnp.cos(bv)], axis=1)

        x = jnp.dot(x0, Wi_ref[...], preferred_element_type=jnp.float32)
        x = _gelu(_ln(x + bi_ref[...], lig_ref[...], lib_ref[...]))

        ti = jax.lax.broadcasted_iota(jnp.int32, (_NP, _T), 1)
        i1 = i2 = None
        for l in range(_L):
            logits = jax.lax.dot_general(
                x, WrT_ref[l], _DNT, preferred_element_type=jnp.float32)
            logits = logits + br_ref[l]
            p = logits - jnp.max(logits, axis=1, keepdims=True)
            p = jnp.exp(p)
            p = p / jnp.sum(p, axis=1, keepdims=True)

            m1 = jnp.max(p, axis=1, keepdims=True)
            i1 = jnp.min(jnp.where(p == m1, ti, _T), axis=1, keepdims=True)
            p_rest = jnp.where(ti == i1, -jnp.inf, p)
            m2 = jnp.max(p_rest, axis=1, keepdims=True)
            i2 = jnp.min(jnp.where(p_rest == m2, ti, _T), axis=1,
                         keepdims=True)
            denom = 1.0 / (m1 + m2 + 1e-9)
            oh = ((ti == i1) | (ti == i2)).astype(jnp.float32)
            gates_full = jnp.where(ti == i1, m1 * denom, 0.0) + jnp.where(
                ti == i2, m2 * denom, 0.0)

            ps_ref[:, l * _T:(l + 1) * _T] = p
            oh_ref[:, l * _T:(l + 1) * _T] = oh

            out = jnp.zeros((_NP, _D), jnp.float32)
            for t in range(_T):
                h = jnp.dot(x, W1_ref[l, t],
                            preferred_element_type=jnp.float32)
                h = _gelu(h + b1_ref[l, t])
                eo = jnp.dot(h, W2_ref[l, t],
                             preferred_element_type=jnp.float32)
                eo = eo + b2_ref[l, t]
                out = out + gates_full[:, t][:, None] * eo
            x = _ln(x + out, lng_ref[l], lnb_ref[l])

        hs = _gelu(jax.lax.dot_general(
            x, Ws1T_ref[...], _DNT, preferred_element_type=jnp.float32)
            + bs1_ref[...])
        sl = jax.lax.dot_general(
            hs, Ws2T_ref[...], _DNT, preferred_element_type=jnp.float32)
        hd = _gelu(jax.lax.dot_general(
            x, Wd1T_ref[...], _DNT, preferred_element_type=jnp.float32)
            + bd1_ref[...])
        dl = jax.lax.dot_general(
            hd, Wd2T_ref[...], _DNT, preferred_element_type=jnp.float32)
        tab = jnp.concatenate(
            [sl + bs2_ref[...], dl + bd2_ref[...],
             i1.astype(jnp.float32), i2.astype(jnp.float32),
             jnp.zeros((_NP, _D - 13), jnp.float32)], axis=1)
        tabT_ref[...] = tab.T

    pair_row = (a_ref[...] * _VR + b_ref[...])[None, :]
    onehotT = (pair_row == jax.lax.broadcasted_iota(
        jnp.int32, (_NP, _BB), 0)).astype(jnp.float32)
    gT = jnp.dot(tabT_ref[0:16], onehotT, preferred_element_type=jnp.float32)
    sumT_ref[...] = gT[0:5, :]
    diffT_ref[...] = gT[5:11, :]
    routT_ref[...] = gT[11:13, :].astype(jnp.int32)
    cnt_ref[...] += jnp.dot(onehotT, jnp.full((_BB, 1), 1.0, jnp.float32),
                             preferred_element_type=jnp.float32)

    @pl.when(i == nsteps - 1)
    def _fin():
        dnc = (((0,), (0,)), ((), ()))
        ps_sum = jax.lax.dot_general(
            cnt_ref[...], ps_ref[...], dnc,
            preferred_element_type=jnp.float32)
        ls_sum = jax.lax.dot_general(
            cnt_ref[...], oh_ref[...], dnc,
            preferred_element_type=jnp.float32)
        scale = np.float32(_T) / np.float32(_B * _B)
        aux_ref[...] = jnp.sum(ps_sum * ls_sum, keepdims=True).reshape(
            1, 1) * scale


@jax.jit
def _run(a, b, params):
    nb = _B // _BB
    p = params

    full = lambda s: pl.BlockSpec(s, lambda i: (0,) * len(s))
    in_specs = [
        pl.BlockSpec((_BB,), lambda i: (i,)),
        pl.BlockSpec((_BB,), lambda i: (i,)),
        full((4 * _NF, _D)), full((_D,)), full((_D,)), full((_D,)),
        full((_L, _T, _D)), full((_L, _T)),
        full((_L, _T, _D, _D)), full((_L, _T, _D)),
        full((_L, _T, _D, _D)), full((_L, _T, _D)),
        full((_L, _D)), full((_L, _D)),
        full((_D // 2, _D)), full((_D // 2,)),
        full((5, _D // 2)), full((5,)),
        full((_D // 2, _D)), full((_D // 2,)),
        full((6, _D // 2)), full((6,)),
    ]
    out_specs = [
        pl.BlockSpec((5, _BB), lambda i: (0, i)),
        pl.BlockSpec((6, _BB), lambda i: (0, i)),
        pl.BlockSpec((_K, _BB), lambda i: (0, i)),
        pl.BlockSpec((1, 1), lambda i: (0, 0)),
    ]
    out_shape = [
        jax.ShapeDtypeStruct((5, _B), jnp.float32),
        jax.ShapeDtypeStruct((6, _B), jnp.float32),
        jax.ShapeDtypeStruct((_K, _B), jnp.int32),
        jax.ShapeDtypeStruct((1, 1), jnp.float32),
    ]
    slT, dlT, routT, aux = pl.pallas_call(
        _net_kernel,
        grid=(nb,),
        in_specs=in_specs,
        out_specs=out_specs,
        out_shape=out_shape,
        scratch_shapes=[
            pltpu.VMEM((_D, _NP), jnp.float32),
            pltpu.VMEM((_NP, _L * _T), jnp.float32),
            pltpu.VMEM((_NP, _L * _T), jnp.float32),
            pltpu.VMEM((_NP, 1), jnp.float32),
        ],
    )(a.astype(jnp.int32), b.astype(jnp.int32),
      p["Wi"], p["bi"], p["ln_in_g"], p["ln_in_b"],
      jnp.swapaxes(p["Wr"], 1, 2), p["br"],
      p["W1"], p["b1"], p["W2"], p["b2"],
      p["ln_g"], p["ln_b"],
      p["Ws1"].T, p["bs1"], p["Ws2"].T, p["bs2"],
      p["Wd1"].T, p["bd1"], p["Wd2"].T, p["bd2"])
    return slT.T, dlT.T, routT.T, aux[0, 0]


def kernel(a, b, params):
    return _run(a, b, params)


# speedup vs baseline: 5.3012x; 1.1515x over previous
import jax
import jax.numpy as jnp
import numpy as np
from jax.experimental import pallas as pl
from jax.experimental.pallas import tpu as pltpu

_B = 8192
_D = 128
_T = 8
_K = 2
_L = 3
_NF = 8
_VR = 16
_NP = _VR * _VR

_DNT = (((1,), (1,)), ((), ()))


def _gelu(x):
    return x * 0.5 * (1.0 + jax.lax.erf(x * np.float32(1.0 / np.sqrt(2.0))))


def _ln(x, g, b):
    m = jnp.mean(x, axis=-1, keepdims=True)
    xc = x - m
    v = jnp.mean(xc * xc, axis=-1, keepdims=True)
    return xc * jax.lax.rsqrt(v + 1e-5) * g + b


def _net_kernel(a_ref, b_ref, Wi_ref, bi_ref, lig_ref, lib_ref,
                WrT_ref, br_ref, W1_ref, b1_ref, W2s_ref, b2_ref,
                lng_ref, lnb_ref, Ws1T_ref, bs1_ref, Ws2T_ref, bs2_ref,
                Wd1T_ref, bd1_ref, Wd2T_ref, bd2_ref,
                sumT_ref, diffT_ref, routT_ref, aux_ref,
                tabT_ref):
    pair = jax.lax.broadcasted_iota(jnp.int32, (_NP, 1), 0)
    pa = (pair // _VR).astype(jnp.float32)
    pb = (pair % _VR).astype(jnp.float32)
    ci = jax.lax.broadcasted_iota(jnp.int32, (_NP, 4 * _NF), 1)
    freqs = jnp.exp2((ci & (_NF - 1)).astype(jnp.float32)) * np.float32(
        2.0 * np.pi / _VR)
    val = jnp.where(ci < 2 * _NF, pa, pb)
    shift = jnp.where((ci & _NF) == _NF, np.float32(np.pi / 2.0),
                      np.float32(0.0))
    x0 = jnp.sin(val * freqs + shift)

    x = jnp.dot(x0, Wi_ref[...], preferred_element_type=jnp.float32)
    x = _gelu(_ln(x + bi_ref[...], lig_ref[...], lib_ref[...]))

    ti = jax.lax.broadcasted_iota(jnp.int32, (_NP, _T), 1)
    expand = (jax.lax.broadcasted_iota(jnp.int32, (_T, _T * _D), 1) // _D
              == jax.lax.broadcasted_iota(jnp.int32, (_T, _T * _D), 0)
              ).astype(jnp.float32)
    i1 = i2 = None
    ps_list = []
    oh_list = []
    for l in range(_L):
        logits = jax.lax.dot_general(
            x, WrT_ref[l], _DNT, preferred_element_type=jnp.float32)
        logits = logits + br_ref[l]
        p = logits - jnp.max(logits, axis=1, keepdims=True)
        p = jnp.exp(p)
        p = p / jnp.sum(p, axis=1, keepdims=True)

        m1 = jnp.max(p, axis=1, keepdims=True)
        i1 = jnp.min(jnp.where(p == m1, ti, _T), axis=1, keepdims=True)
        p_rest = jnp.where(ti == i1, -jnp.inf, p)
        m2 = jnp.max(p_rest, axis=1, keepdims=True)
        i2 = jnp.min(jnp.where(p_rest == m2, ti, _T), axis=1, keepdims=True)
        denom = 1.0 / (m1 + m2 + 1e-9)
        oh = ((ti == i1) | (ti == i2)).astype(jnp.float32)
        gates_full = jnp.where(ti == i1, m1 * denom, 0.0) + jnp.where(
            ti == i2, m2 * denom, 0.0)
        ps_list.append(p)
        oh_list.append(oh)

        h_all = jnp.concatenate(
            [jnp.dot(x, W1_ref[l, t], preferred_element_type=jnp.float32)
             + b1_ref[l, t] for t in range(_T)], axis=1)
        h_all = _gelu(h_all)
        gw = jnp.dot(gates_full, expand, preferred_element_type=jnp.float32)
        out = jnp.dot(h_all * gw, W2s_ref[l],
                      preferred_element_type=jnp.float32)
        out = out + jnp.dot(gates_full, b2_ref[l],
                            preferred_element_type=jnp.float32)
        x = _ln(x + out, lng_ref[l], lnb_ref[l])

    hs = _gelu(jax.lax.dot_general(
        x, Ws1T_ref[...], _DNT, preferred_element_type=jnp.float32)
        + bs1_ref[...])
    sl = jax.lax.dot_general(
        hs, Ws2T_ref[...], _DNT, preferred_element_type=jnp.float32)
    hd = _gelu(jax.lax.dot_general(
        x, Wd1T_ref[...], _DNT, preferred_element_type=jnp.float32)
        + bd1_ref[...])
    dl = jax.lax.dot_general(
        hd, Wd2T_ref[...], _DNT, preferred_element_type=jnp.float32)
    tab = jnp.concatenate(
        [sl + bs2_ref[...], dl + bd2_ref[...],
         i1.astype(jnp.float32), i2.astype(jnp.float32),
         jnp.zeros((_NP, 3), jnp.float32)] + ps_list + oh_list, axis=1)
    tabT_ref[...] = tab.T

    pair_row = (a_ref[...] * _VR + b_ref[...])[None, :]
    onehotT = (pair_row == jax.lax.broadcasted_iota(
        jnp.int32, (_NP, _B), 0)).astype(jnp.float32)
    gT = jnp.dot(tabT_ref[...], onehotT, preferred_element_type=jnp.float32)
    sumT_ref[...] = gT[0:5, :]
    diffT_ref[...] = gT[5:11, :]
    routT_ref[...] = gT[11:13, :].astype(jnp.int32)

    ps_sum = jnp.sum(gT[16:16 + _L * _T, :], axis=1, keepdims=True)
    ls_sum = jnp.sum(gT[16 + _L * _T:16 + 2 * _L * _T, :], axis=1,
                     keepdims=True)
    scale = np.float32(_T) / np.float32(_B * _B)
    aux_ref[...] = jnp.sum(ps_sum * ls_sum, keepdims=True).reshape(
        1, 1) * scale


@jax.jit
def _run(a, b, params):
    p = params

    full = lambda s: pl.BlockSpec(s, lambda: (0,) * len(s))
    in_specs = [
        full((_B,)), full((_B,)),
        full((4 * _NF, _D)), full((_D,)), full((_D,)), full((_D,)),
        full((_L, _T, _D)), full((_L, _T)),
        full((_L, _T, _D, _D)), full((_L, _T, _D)),
        full((_L, _T * _D, _D)), full((_L, _T, _D)),
        full((_L, _D)), full((_L, _D)),
        full((_D // 2, _D)), full((_D // 2,)),
        full((5, _D // 2)), full((5,)),
        full((_D // 2, _D)), full((_D // 2,)),
        full((6, _D // 2)), full((6,)),
    ]
    out_specs = [
        full((5, _B)), full((6, _B)), full((_K, _B)), full((1, 1)),
    ]
    out_shape = [
        jax.ShapeDtypeStruct((5, _B), jnp.float32),
        jax.ShapeDtypeStruct((6, _B), jnp.float32),
        jax.ShapeDtypeStruct((_K, _B), jnp.int32),
        jax.ShapeDtypeStruct((1, 1), jnp.float32),
    ]
    slT, dlT, routT, aux = pl.pallas_call(
        _net_kernel,
        in_specs=in_specs,
        out_specs=out_specs,
        out_shape=out_shape,
        scratch_shapes=[
            pltpu.VMEM((16 + 2 * _L * _T, _NP), jnp.float32),
        ],
    )(a.astype(jnp.int32), b.astype(jnp.int32),
      p["Wi"], p["bi"], p["ln_in_g"], p["ln_in_b"],
      jnp.swapaxes(p["Wr"], 1, 2), p["br"],
      p["W1"], p["b1"], p["W2"].reshape(_L, _T * _D, _D), p["b2"],
      p["ln_g"], p["ln_b"],
      p["Ws1"].T, p["bs1"], p["Ws2"].T, p["bs2"],
      p["Wd1"].T, p["bd1"], p["Wd2"].T, p["bd2"])
    return slT.T, dlT.T, routT.T, aux[0, 0]


def kernel(a, b, params):
    return _run(a, b, params)
